# baseline (speedup 1.0000x reference)
import functools

import jax
import jax.numpy as jnp
from jax import lax
from jax.experimental import pallas as pl
from jax.experimental.pallas import tpu as pltpu
from jax.experimental.pallas import tpu_sc as plsc

_PRIMES = (100003, 100019, 100043, 100049,
           100057, 100069, 100103, 100109)
_HEAD_DIM = 16
_TOKENIZER_VOCAB = 128000
_B = 4
_T = 2048
_TOK = _B * _T
_NW = 32
_CHUNK = _TOK // _NW
_LANES = 16
_GROUPS = _CHUNK // _LANES
_WIN = _CHUNK + 16
_CV = 100112
_RING = 4

_R8 = tuple(tuple(pow(2, 8 * k, p) for k in range(8)) for p in _PRIMES)


def _i32(v):
    return jnp.int32(v)


def _srl(x, k):
    return lax.shift_right_logical(x, jnp.int32(k))


def _prod_limbs(a, m):
    a0 = a & 0xFFFF
    a1 = _srl(a, 16)
    t = a0 * m[0]
    l0 = t & 0xFFFF
    c = _srl(t, 16)
    t = a0 * m[1] + a1 * m[0] + c
    l1 = t & 0xFFFF
    c = _srl(t, 16)
    t = a0 * m[2] + a1 * m[1] + c
    l2 = t & 0xFFFF
    c = _srl(t, 16)
    t = a0 * m[3] + a1 * m[2] + c
    l3 = t & 0xFFFF
    return (l0, l1, l2, l3)


def _chunks8(limbs):
    out = []
    for l in limbs:
        out.append(l & 0xFF)
        out.append(_srl(l, 8))
    return out


def _mod_p(chunks, h):
    p = _PRIMES[h]
    r8 = _R8[h]
    s = chunks[0] * r8[0]
    for k in range(1, 8):
        s = s + chunks[k] * r8[k]
    q = (s.astype(jnp.float32) * jnp.float32(1.0 / p)).astype(jnp.int32)
    r = s - q * p
    r = jnp.where(r < 0, r + p, r)
    r = jnp.where(r >= p, r - p, r)
    return r


def _engram_body(inp, lut, mlv_hbm, ctab,
                 out, raw_a, raw_b, comp, hidx, ring, obuf, mlv,
                 sem, rs0, rs1, rs2, rs3):
    ring_sems = (rs0, rs1, rs2, rs3)
    wid = lax.axis_index("s") * 2 + lax.axis_index("c")
    base = wid * _CHUNK
    start = base - 16

    pltpu.sync_copy(mlv_hbm, mlv)

    @pl.when(wid == 0)
    def _():
        raw_a[_i32(0), pl.ds(_i32(0), 16)] = jnp.zeros((16,), jnp.int32)
        pltpu.sync_copy(inp.at[pl.ds(_i32(0), 112)], raw_a.at[_i32(0), pl.ds(_i32(16), 112)])
        pltpu.sync_copy(inp.at[pl.ds(_i32(112), 128)], raw_a.at[_i32(1)])
        pltpu.sync_copy(inp.at[pl.ds(_i32(240), 16)], raw_b)

    @pl.when(wid > 0)
    def _():
        pltpu.sync_copy(inp.at[pl.ds(start, 128)], raw_a.at[_i32(0)])
        pltpu.sync_copy(inp.at[pl.ds(start + 128, 128)], raw_a.at[_i32(1)])
        pltpu.sync_copy(inp.at[pl.ds(start + 256, 16)], raw_b)

    for r in range(2):
        for j in range(8):
            sl = pl.ds(_i32(j * 16), 16)
            raw_a[_i32(r), sl] = jnp.clip(raw_a[_i32(r), sl], 0, _TOKENIZER_VOCAB - 1)
    raw_b[...] = jnp.clip(raw_b[...], 0, _TOKENIZER_VOCAB - 1)

    g1 = pltpu.async_copy(lut.at[raw_a.at[_i32(0)]], comp.at[pl.ds(_i32(0), 128)], sem)
    g2 = pltpu.async_copy(lut.at[raw_a.at[_i32(1)]], comp.at[pl.ds(_i32(128), 128)], sem)
    g3 = pltpu.async_copy(lut.at[raw_b], comp.at[pl.ds(_i32(256), 16)], sem)
    g1.wait()
    g2.wait()
    g3.wait()

    m0 = tuple(mlv[_i32(k)] for k in range(4))
    m1 = tuple(mlv[_i32(4 + k)] for k in range(4))
    m2 = tuple(mlv[_i32(8 + k)] for k in range(4))
    rowpos = (base & (_T - 1)) + lax.iota(jnp.int32, 16)

    for g in range(_GROUPS):
        off = 16 + g * 16
        s0v = comp[pl.ds(_i32(off), 16)]
        s1v = comp[pl.ds(_i32(off - 1), 16)]
        s2v = comp[pl.ds(_i32(off - 2), 16)]
        if g == 0:
            s1v = jnp.where(rowpos >= 1, s1v, 0)
            s2v = jnp.where(rowpos >= 2, s2v, 0)
        p0 = _prod_limbs(s0v, m0)
        p1 = _prod_limbs(s1v, m1)
        p2 = _prod_limbs(s2v, m2)
        mix2 = tuple(x ^ y for x, y in zip(p0, p1))
        mix3 = tuple(x ^ y for x, y in zip(mix2, p2))
        c2 = _chunks8(mix2)
        c3 = _chunks8(mix3)
        dst = pl.ds(_i32((g % 8) * 16), 16)
        for h in range(8):
            hidx[_i32(h), _i32(g // 8), dst] = _mod_p(c2 if h < 4 else c3, h)

    def fire(i):
        c, h = divmod(i, 8)
        r = i % _RING
        return pltpu.async_copy(
            ctab.at[hidx.at[_i32(h), _i32(c)]],
            ring.at[_i32(r)], ring_sems[r])

    def band_copy(i):
        c, h = divmod(i, 8)
        r = i % _RING
        band = pl.ds(_i32(h * _HEAD_DIM), _HEAD_DIM)

        def body(t, carry):
            obuf[_i32(c), t, band] = ring[_i32(r), t, band]
            return carry

        lax.fori_loop(0, 128, body, jnp.int32(0), unroll=4)

    def drain(entry):
        i, cp = entry
        cp.wait()
        band_copy(i)
        if i % 8 == 7:
            c = i // 8
            return pltpu.async_copy(
                obuf.at[_i32(c)],
                out.at[pl.ds(base + c * 128, 128)], sem)
        return None

    inflight = []
    writes = []
    for i in range(16):
        if len(inflight) == _RING:
            w = drain(inflight.pop(0))
            if w is not None:
                writes.append(w)
        inflight.append((i, fire(i)))
    while inflight:
        w = drain(inflight.pop(0))
        if w is not None:
            writes.append(w)
    for w in writes:
        w.wait()


@jax.jit
def _engram_call(inp, lut, mlimbs, ctab):
    mesh = plsc.VectorSubcoreMesh(core_axis_name="c", subcore_axis_name="s")
    f = functools.partial(
        pl.kernel,
        mesh=mesh,
        out_type=jax.ShapeDtypeStruct((_TOK, 8 * _HEAD_DIM), jnp.float32),
        scratch_types=[
            pltpu.VMEM((2, 128), jnp.int32),
            pltpu.VMEM((16,), jnp.int32),
            pltpu.VMEM((_WIN,), jnp.int32),
            pltpu.VMEM((8, 2, 128), jnp.int32),
            pltpu.VMEM((_RING, 128, 128), jnp.float32),
            pltpu.VMEM((2, 128, 128), jnp.float32),
            pltpu.VMEM((12, 16), jnp.int32),
            pltpu.SemaphoreType.DMA,
            pltpu.SemaphoreType.DMA,
            pltpu.SemaphoreType.DMA,
            pltpu.SemaphoreType.DMA,
            pltpu.SemaphoreType.DMA,
        ],
    )(_engram_body)
    return f(inp, lut, mlimbs, ctab)


def kernel(input_ids, lookup_table, multipliers,
           table_0, table_1, table_2, table_3,
           table_4, table_5, table_6, table_7):
    tables = (table_0, table_1, table_2, table_3,
              table_4, table_5, table_6, table_7)
    inp = input_ids.reshape(-1).astype(jnp.int32)
    lut = lookup_table.astype(jnp.int32)
    shifts = jnp.asarray([0, 16, 32, 48], dtype=multipliers.dtype)
    limbs = ((multipliers[:, None] >> shifts[None, :]) & 0xFFFF).astype(jnp.int32)
    mlimbs = jnp.broadcast_to(limbs.reshape(12, 1), (12, 16))
    ctab = jnp.concatenate(
        [jnp.pad(t, ((0, _CV - t.shape[0]), (0, 0))) for t in tables], axis=1)
    out = _engram_call(inp, lut, mlimbs, ctab)
    return out.reshape(_B, _T, 8 * _HEAD_DIM)


# speedup vs baseline: 1.0899x; 1.0899x over previous
import functools

import jax
import jax.numpy as jnp
from jax import lax
from jax.experimental import pallas as pl
from jax.experimental.pallas import tpu as pltpu
from jax.experimental.pallas import tpu_sc as plsc

_PRIMES = (100003, 100019, 100043, 100049,
           100057, 100069, 100103, 100109)
_HEAD_DIM = 16
_TOKENIZER_VOCAB = 128000
_B = 4
_T = 2048
_TOK = _B * _T
_NW = 32
_CHUNK = _TOK // _NW
_LANES = 16
_GROUPS = _CHUNK // _LANES
_WIN = _CHUNK + 16
_CV = 100224
_RING = 4

_R8 = tuple(tuple(pow(2, 8 * k, p) for k in range(8)) for p in _PRIMES)


def _i32(v):
    return jnp.int32(v)


def _srl(x, k):
    return lax.shift_right_logical(x, jnp.int32(k))


def _prod_limbs(a, m):
    a0 = a & 0xFFFF
    a1 = _srl(a, 16)
    t = a0 * m[0]
    l0 = t & 0xFFFF
    c = _srl(t, 16)
    t = a0 * m[1] + a1 * m[0] + c
    l1 = t & 0xFFFF
    c = _srl(t, 16)
    t = a0 * m[2] + a1 * m[1] + c
    l2 = t & 0xFFFF
    c = _srl(t, 16)
    t = a0 * m[3] + a1 * m[2] + c
    l3 = t & 0xFFFF
    return (l0, l1, l2, l3)


def _chunks8(limbs):
    out = []
    for l in limbs:
        out.append(l & 0xFF)
        out.append(_srl(l, 8))
    return out


def _mod_p(chunks, h):
    p = _PRIMES[h]
    r8 = _R8[h]
    s = chunks[0] * r8[0]
    for k in range(1, 8):
        s = s + chunks[k] * r8[k]
    q = (s.astype(jnp.float32) * jnp.float32(1.0 / p)).astype(jnp.int32)
    r = s - q * p
    r = jnp.where(r < 0, r + p, r)
    r = jnp.where(r >= p, r - p, r)
    return r


def _engram_body(inp, lut, mlv_hbm, ctab,
                 out, raw_a, raw_b, comp, hidx, ring, obuf, mlv,
                 sem, rs0, rs1, rs2, rs3):
    ring_sems = (rs0, rs1, rs2, rs3)
    wid = lax.axis_index("s") * 2 + lax.axis_index("c")
    base = wid * _CHUNK
    start = base - 16

    pltpu.sync_copy(mlv_hbm, mlv)

    @pl.when(wid == 0)
    def _():
        raw_a[_i32(0), pl.ds(_i32(0), 16)] = jnp.zeros((16,), jnp.int32)
        pltpu.sync_copy(inp.at[pl.ds(_i32(0), 112)], raw_a.at[_i32(0), pl.ds(_i32(16), 112)])
        pltpu.sync_copy(inp.at[pl.ds(_i32(112), 128)], raw_a.at[_i32(1)])
        pltpu.sync_copy(inp.at[pl.ds(_i32(240), 16)], raw_b)

    @pl.when(wid > 0)
    def _():
        pltpu.sync_copy(inp.at[pl.ds(start, 128)], raw_a.at[_i32(0)])
        pltpu.sync_copy(inp.at[pl.ds(start + 128, 128)], raw_a.at[_i32(1)])
        pltpu.sync_copy(inp.at[pl.ds(start + 256, 16)], raw_b)

    for r in range(2):
        for j in range(8):
            sl = pl.ds(_i32(j * 16), 16)
            raw_a[_i32(r), sl] = jnp.clip(raw_a[_i32(r), sl], 0, _TOKENIZER_VOCAB - 1)
    raw_b[...] = jnp.clip(raw_b[...], 0, _TOKENIZER_VOCAB - 1)

    g1 = pltpu.async_copy(lut.at[raw_a.at[_i32(0)]], comp.at[pl.ds(_i32(0), 128)], sem)
    g2 = pltpu.async_copy(lut.at[raw_a.at[_i32(1)]], comp.at[pl.ds(_i32(128), 128)], sem)
    g3 = pltpu.async_copy(lut.at[raw_b], comp.at[pl.ds(_i32(256), 16)], sem)
    g1.wait()
    g2.wait()
    g3.wait()

    m0 = tuple(mlv[_i32(k)] for k in range(4))
    m1 = tuple(mlv[_i32(4 + k)] for k in range(4))
    m2 = tuple(mlv[_i32(8 + k)] for k in range(4))
    rowpos = (base & (_T - 1)) + lax.iota(jnp.int32, 16)

    for g in range(_GROUPS):
        off = 16 + g * 16
        s0v = comp[pl.ds(_i32(off), 16)]
        s1v = comp[pl.ds(_i32(off - 1), 16)]
        s2v = comp[pl.ds(_i32(off - 2), 16)]
        if g == 0:
            s1v = jnp.where(rowpos >= 1, s1v, 0)
            s2v = jnp.where(rowpos >= 2, s2v, 0)
        p0 = _prod_limbs(s0v, m0)
        p1 = _prod_limbs(s1v, m1)
        p2 = _prod_limbs(s2v, m2)
        mix2 = tuple(x ^ y for x, y in zip(p0, p1))
        mix3 = tuple(x ^ y for x, y in zip(mix2, p2))
        c2 = _chunks8(mix2)
        c3 = _chunks8(mix3)
        dst = pl.ds(_i32((g % 8) * 16), 16)
        for h in range(8):
            hidx[_i32(h), _i32(g // 8), dst] = _mod_p(c2 if h < 4 else c3, h)

    def fire(i):
        c, h = divmod(i, 8)
        r = i % _RING
        return pltpu.async_copy(
            ctab.at[hidx.at[_i32(h), _i32(c)]],
            ring.at[_i32(r)], ring_sems[r])

    def band_copy(i):
        c, h = divmod(i, 8)
        r = i % _RING
        band = pl.ds(_i32(h * _HEAD_DIM), _HEAD_DIM)

        def body(t, carry):
            obuf[_i32(c), t, band] = ring[_i32(r), t, band]
            return carry

        lax.fori_loop(0, 128, body, jnp.int32(0), unroll=4)

    def drain(entry):
        i, cp = entry
        cp.wait()
        band_copy(i)
        if i % 8 == 7:
            c = i // 8
            return pltpu.async_copy(
                obuf.at[_i32(c)],
                out.at[pl.ds(base + c * 128, 128)], sem)
        return None

    inflight = []
    writes = []
    for i in range(16):
        if len(inflight) == _RING:
            w = drain(inflight.pop(0))
            if w is not None:
                writes.append(w)
        inflight.append((i, fire(i)))
    while inflight:
        w = drain(inflight.pop(0))
        if w is not None:
            writes.append(w)
    for w in writes:
        w.wait()


def _ctab_body(*refs):
    outs = refs[8]
    for h in range(8):
        outs[:, h * _HEAD_DIM:(h + 1) * _HEAD_DIM] = jnp.transpose(
            refs[h][...], (1, 0))


def _build_ctab(*tts):
    grid = _CV // 128
    in_specs = [pl.BlockSpec((16, 128), lambda j: (jnp.int32(0), j)) for _ in range(8)]
    out_specs = pl.BlockSpec((128, 128), lambda j: (j, jnp.int32(0)))
    return pl.pallas_call(
        _ctab_body,
        grid=(grid,),
        in_specs=in_specs,
        out_specs=out_specs,
        out_shape=jax.ShapeDtypeStruct((_CV, 128), jnp.float32),
    )(*tts)


@jax.jit
def _engram_call(inp, lut, mlimbs, ctab):
    mesh = plsc.VectorSubcoreMesh(core_axis_name="c", subcore_axis_name="s")
    f = functools.partial(
        pl.kernel,
        mesh=mesh,
        out_type=jax.ShapeDtypeStruct((_TOK, 8 * _HEAD_DIM), jnp.float32),
        scratch_types=[
            pltpu.VMEM((2, 128), jnp.int32),
            pltpu.VMEM((16,), jnp.int32),
            pltpu.VMEM((_WIN,), jnp.int32),
            pltpu.VMEM((8, 2, 128), jnp.int32),
            pltpu.VMEM((_RING, 128, 128), jnp.float32),
            pltpu.VMEM((2, 128, 128), jnp.float32),
            pltpu.VMEM((12, 16), jnp.int32),
            pltpu.SemaphoreType.DMA,
            pltpu.SemaphoreType.DMA,
            pltpu.SemaphoreType.DMA,
            pltpu.SemaphoreType.DMA,
            pltpu.SemaphoreType.DMA,
        ],
    )(_engram_body)
    return f(inp, lut, mlimbs, ctab)


def kernel(input_ids, lookup_table, multipliers,
           table_0, table_1, table_2, table_3,
           table_4, table_5, table_6, table_7):
    tables = (table_0, table_1, table_2, table_3,
              table_4, table_5, table_6, table_7)
    inp = input_ids.reshape(-1).astype(jnp.int32)
    lut = lookup_table.astype(jnp.int32)
    shifts = jnp.asarray([0, 16, 32, 48], dtype=multipliers.dtype)
    limbs = ((multipliers[:, None] >> shifts[None, :]) & 0xFFFF).astype(jnp.int32)
    mlimbs = jnp.broadcast_to(limbs.reshape(12, 1), (12, 16))
    ctab = _build_ctab(*(t.T for t in tables))
    out = _engram_call(inp, lut, mlimbs, ctab)
    return out.reshape(_B, _T, 8 * _HEAD_DIM)


# speedup vs baseline: 3.4467x; 3.1624x over previous
import functools

import jax
import jax.numpy as jnp
from jax import lax
from jax.experimental import pallas as pl
from jax.experimental.pallas import tpu as pltpu
from jax.experimental.pallas import tpu_sc as plsc

_PRIMES = (100003, 100019, 100043, 100049,
           100057, 100069, 100103, 100109)
_HEAD_DIM = 16
_TOKENIZER_VOCAB = 128000
_B = 4
_T = 2048
_TOK = _B * _T
_NW = 32
_CHUNK = _TOK // _NW
_LANES = 16
_GROUPS = _CHUNK // _LANES
_WIN = _CHUNK + 16
_CV = 100352
_RING = 4

_R8 = tuple(tuple(pow(2, 8 * k, p) for k in range(8)) for p in _PRIMES)


def _i32(v):
    return jnp.int32(v)


def _srl(x, k):
    return lax.shift_right_logical(x, jnp.int32(k))


def _prod_limbs(a, m):
    a0 = a & 0xFFFF
    a1 = _srl(a, 16)
    t = a0 * m[0]
    l0 = t & 0xFFFF
    c = _srl(t, 16)
    t = a0 * m[1] + a1 * m[0] + c
    l1 = t & 0xFFFF
    c = _srl(t, 16)
    t = a0 * m[2] + a1 * m[1] + c
    l2 = t & 0xFFFF
    c = _srl(t, 16)
    t = a0 * m[3] + a1 * m[2] + c
    l3 = t & 0xFFFF
    return (l0, l1, l2, l3)


def _chunks8(limbs):
    out = []
    for l in limbs:
        out.append(l & 0xFF)
        out.append(_srl(l, 8))
    return out


def _mod_p(chunks, h):
    p = _PRIMES[h]
    r8 = _R8[h]
    s = chunks[0] * r8[0]
    for k in range(1, 8):
        s = s + chunks[k] * r8[k]
    q = (s.astype(jnp.float32) * jnp.float32(1.0 / p)).astype(jnp.int32)
    r = s - q * p
    r = jnp.where(r < 0, r + p, r)
    r = jnp.where(r >= p, r - p, r)
    return r


def _engram_body(inp, lut, mlv_hbm, ctab,
                 out, raw_a, raw_b, comp, hidx, ring, obuf, mlv,
                 sem, rs0, rs1, rs2, rs3):
    ring_sems = (rs0, rs1, rs2, rs3)
    wid = lax.axis_index("s") * 2 + lax.axis_index("c")
    base = wid * _CHUNK
    start = base - 16

    pltpu.sync_copy(mlv_hbm, mlv)

    @pl.when(wid == 0)
    def _():
        raw_a[_i32(0), pl.ds(_i32(0), 16)] = jnp.zeros((16,), jnp.int32)
        pltpu.sync_copy(inp.at[pl.ds(_i32(0), 112)], raw_a.at[_i32(0), pl.ds(_i32(16), 112)])
        pltpu.sync_copy(inp.at[pl.ds(_i32(112), 128)], raw_a.at[_i32(1)])
        pltpu.sync_copy(inp.at[pl.ds(_i32(240), 16)], raw_b)

    @pl.when(wid > 0)
    def _():
        pltpu.sync_copy(inp.at[pl.ds(start, 128)], raw_a.at[_i32(0)])
        pltpu.sync_copy(inp.at[pl.ds(start + 128, 128)], raw_a.at[_i32(1)])
        pltpu.sync_copy(inp.at[pl.ds(start + 256, 16)], raw_b)

    for r in range(2):
        for j in range(8):
            sl = pl.ds(_i32(j * 16), 16)
            raw_a[_i32(r), sl] = jnp.clip(raw_a[_i32(r), sl], 0, _TOKENIZER_VOCAB - 1)
    raw_b[...] = jnp.clip(raw_b[...], 0, _TOKENIZER_VOCAB - 1)

    g1 = pltpu.async_copy(lut.at[raw_a.at[_i32(0)]], comp.at[pl.ds(_i32(0), 128)], sem)
    g2 = pltpu.async_copy(lut.at[raw_a.at[_i32(1)]], comp.at[pl.ds(_i32(128), 128)], sem)
    g3 = pltpu.async_copy(lut.at[raw_b], comp.at[pl.ds(_i32(256), 16)], sem)
    g1.wait()
    g2.wait()
    g3.wait()

    m0 = tuple(mlv[_i32(k)] for k in range(4))
    m1 = tuple(mlv[_i32(4 + k)] for k in range(4))
    m2 = tuple(mlv[_i32(8 + k)] for k in range(4))
    rowpos = (base & (_T - 1)) + lax.iota(jnp.int32, 16)

    for g in range(_GROUPS):
        off = 16 + g * 16
        s0v = comp[pl.ds(_i32(off), 16)]
        s1v = comp[pl.ds(_i32(off - 1), 16)]
        s2v = comp[pl.ds(_i32(off - 2), 16)]
        if g == 0:
            s1v = jnp.where(rowpos >= 1, s1v, 0)
            s2v = jnp.where(rowpos >= 2, s2v, 0)
        p0 = _prod_limbs(s0v, m0)
        p1 = _prod_limbs(s1v, m1)
        p2 = _prod_limbs(s2v, m2)
        mix2 = tuple(x ^ y for x, y in zip(p0, p1))
        mix3 = tuple(x ^ y for x, y in zip(mix2, p2))
        c2 = _chunks8(mix2)
        c3 = _chunks8(mix3)
        dst = pl.ds(_i32((g % 8) * 16), 16)
        for h in range(8):
            hidx[_i32(h), _i32(g // 8), dst] = _mod_p(c2 if h < 4 else c3, h)

    def fire(i):
        c, h = divmod(i, 8)
        r = i % _RING
        return pltpu.async_copy(
            ctab.at[hidx.at[_i32(h), _i32(c)]],
            ring.at[_i32(r)], ring_sems[r])

    def band_copy(i):
        c, h = divmod(i, 8)
        r = i % _RING
        band = pl.ds(_i32(h * _HEAD_DIM), _HEAD_DIM)

        def body(t, carry):
            obuf[_i32(c), t, band] = ring[_i32(r), t, band]
            return carry

        lax.fori_loop(0, 128, body, jnp.int32(0), unroll=4)

    def drain(entry):
        i, cp = entry
        cp.wait()
        band_copy(i)
        if i % 8 == 7:
            c = i // 8
            return pltpu.async_copy(
                obuf.at[_i32(c)],
                out.at[pl.ds(base + c * 128, 128)], sem)
        return None

    inflight = []
    writes = []
    for i in range(16):
        if len(inflight) == _RING:
            w = drain(inflight.pop(0))
            if w is not None:
                writes.append(w)
        inflight.append((i, fire(i)))
    while inflight:
        w = drain(inflight.pop(0))
        if w is not None:
            writes.append(w)
    for w in writes:
        w.wait()


def _ctab_body(*refs):
    x = jnp.concatenate([refs[h][...] for h in range(8)], axis=0)
    refs[8][...] = jnp.transpose(x, (1, 0))


def _build_ctab(*tts):
    grid = _CV // 512
    in_specs = [pl.BlockSpec((16, 512), lambda j: (jnp.int32(0), j)) for _ in range(8)]
    out_specs = pl.BlockSpec((512, 128), lambda j: (j, jnp.int32(0)))
    return pl.pallas_call(
        _ctab_body,
        grid=(grid,),
        in_specs=in_specs,
        out_specs=out_specs,
        out_shape=jax.ShapeDtypeStruct((_CV, 128), jnp.float32),
    )(*tts)


@jax.jit
def _engram_call(inp, lut, mlimbs, ctab):
    mesh = plsc.VectorSubcoreMesh(core_axis_name="c", subcore_axis_name="s")
    f = functools.partial(
        pl.kernel,
        mesh=mesh,
        out_type=jax.ShapeDtypeStruct((_TOK, 8 * _HEAD_DIM), jnp.float32),
        scratch_types=[
            pltpu.VMEM((2, 128), jnp.int32),
            pltpu.VMEM((16,), jnp.int32),
            pltpu.VMEM((_WIN,), jnp.int32),
            pltpu.VMEM((8, 2, 128), jnp.int32),
            pltpu.VMEM((_RING, 128, 128), jnp.float32),
            pltpu.VMEM((2, 128, 128), jnp.float32),
            pltpu.VMEM((12, 16), jnp.int32),
            pltpu.SemaphoreType.DMA,
            pltpu.SemaphoreType.DMA,
            pltpu.SemaphoreType.DMA,
            pltpu.SemaphoreType.DMA,
            pltpu.SemaphoreType.DMA,
        ],
    )(_engram_body)
    return f(inp, lut, mlimbs, ctab)


def kernel(input_ids, lookup_table, multipliers,
           table_0, table_1, table_2, table_3,
           table_4, table_5, table_6, table_7):
    tables = (table_0, table_1, table_2, table_3,
              table_4, table_5, table_6, table_7)
    inp = input_ids.reshape(-1).astype(jnp.int32)
    lut = lookup_table.astype(jnp.int32)
    shifts = jnp.asarray([0, 16, 32, 48], dtype=multipliers.dtype)
    limbs = ((multipliers[:, None] >> shifts[None, :]) & 0xFFFF).astype(jnp.int32)
    mlimbs = jnp.broadcast_to(limbs.reshape(12, 1), (12, 16))
    ctab = _build_ctab(*(t.T for t in tables))
    out = _engram_call(inp, lut, mlimbs, ctab)
    return out.reshape(_B, _T, 8 * _HEAD_DIM)


# speedup vs baseline: 5.9564x; 1.7282x over previous
import functools

import jax
import jax.numpy as jnp
from jax import lax
from jax.experimental import pallas as pl
from jax.experimental.pallas import tpu as pltpu
from jax.experimental.pallas import tpu_sc as plsc

_PRIMES = (100003, 100019, 100043, 100049,
           100057, 100069, 100103, 100109)
_HEAD_DIM = 16
_TOKENIZER_VOCAB = 128000
_B = 4
_T = 2048
_TOK = _B * _T
_NW = 32
_CHUNK = _TOK // _NW
_LANES = 16
_GROUPS = _CHUNK // _LANES
_WIN = _CHUNK + 16
_CV = 100352
_RING = 4

_R8 = tuple(tuple(pow(2, 8 * k, p) for k in range(8)) for p in _PRIMES)


def _i32(v):
    return jnp.int32(v)


def _srl(x, k):
    return lax.shift_right_logical(x, jnp.int32(k))


def _prod_limbs(a, m):
    a0 = a & 0xFFFF
    a1 = _srl(a, 16)
    t = a0 * m[0]
    l0 = t & 0xFFFF
    c = _srl(t, 16)
    t = a0 * m[1] + a1 * m[0] + c
    l1 = t & 0xFFFF
    c = _srl(t, 16)
    t = a0 * m[2] + a1 * m[1] + c
    l2 = t & 0xFFFF
    c = _srl(t, 16)
    t = a0 * m[3] + a1 * m[2] + c
    l3 = t & 0xFFFF
    return (l0, l1, l2, l3)


def _chunks8(limbs):
    out = []
    for l in limbs:
        out.append(l & 0xFF)
        out.append(_srl(l, 8))
    return out


def _mod_p(chunks, h):
    p = _PRIMES[h]
    r8 = _R8[h]
    s = chunks[0] * r8[0]
    for k in range(1, 8):
        s = s + chunks[k] * r8[k]
    q = (s.astype(jnp.float32) * jnp.float32(1.0 / p)).astype(jnp.int32)
    r = s - q * p
    r = jnp.where(r < 0, r + p, r)
    r = jnp.where(r >= p, r - p, r)
    return r


def _engram_body(inp, lut, mlv_hbm, ctab,
                 out, raw_a, raw_b, comp, hidx, ring, obuf, mlv,
                 sem, rs0, rs1, rs2, rs3):
    ring_sems = (rs0, rs1, rs2, rs3)
    wid = lax.axis_index("s") * 2 + lax.axis_index("c")
    base = wid * _CHUNK
    start = base - 16

    pltpu.sync_copy(mlv_hbm, mlv)

    @pl.when(wid == 0)
    def _():
        raw_a[_i32(0), pl.ds(_i32(0), 16)] = jnp.zeros((16,), jnp.int32)
        pltpu.sync_copy(inp.at[pl.ds(_i32(0), 112)], raw_a.at[_i32(0), pl.ds(_i32(16), 112)])
        pltpu.sync_copy(inp.at[pl.ds(_i32(112), 128)], raw_a.at[_i32(1)])
        pltpu.sync_copy(inp.at[pl.ds(_i32(240), 16)], raw_b)

    @pl.when(wid > 0)
    def _():
        pltpu.sync_copy(inp.at[pl.ds(start, 128)], raw_a.at[_i32(0)])
        pltpu.sync_copy(inp.at[pl.ds(start + 128, 128)], raw_a.at[_i32(1)])
        pltpu.sync_copy(inp.at[pl.ds(start + 256, 16)], raw_b)

    for r in range(2):
        for j in range(8):
            sl = pl.ds(_i32(j * 16), 16)
            raw_a[_i32(r), sl] = jnp.clip(raw_a[_i32(r), sl], 0, _TOKENIZER_VOCAB - 1)
    raw_b[...] = jnp.clip(raw_b[...], 0, _TOKENIZER_VOCAB - 1)

    g1 = pltpu.async_copy(lut.at[raw_a.at[_i32(0)]], comp.at[pl.ds(_i32(0), 128)], sem)
    g2 = pltpu.async_copy(lut.at[raw_a.at[_i32(1)]], comp.at[pl.ds(_i32(128), 128)], sem)
    g3 = pltpu.async_copy(lut.at[raw_b], comp.at[pl.ds(_i32(256), 16)], sem)
    g1.wait()
    g2.wait()
    g3.wait()

    m0 = tuple(mlv[_i32(k)] for k in range(4))
    m1 = tuple(mlv[_i32(4 + k)] for k in range(4))
    m2 = tuple(mlv[_i32(8 + k)] for k in range(4))
    rowpos = (base & (_T - 1)) + lax.iota(jnp.int32, 16)

    for g in range(_GROUPS):
        off = 16 + g * 16
        s0v = comp[pl.ds(_i32(off), 16)]
        s1v = comp[pl.ds(_i32(off - 1), 16)]
        s2v = comp[pl.ds(_i32(off - 2), 16)]
        if g == 0:
            s1v = jnp.where(rowpos >= 1, s1v, 0)
            s2v = jnp.where(rowpos >= 2, s2v, 0)
        p0 = _prod_limbs(s0v, m0)
        p1 = _prod_limbs(s1v, m1)
        p2 = _prod_limbs(s2v, m2)
        mix2 = tuple(x ^ y for x, y in zip(p0, p1))
        mix3 = tuple(x ^ y for x, y in zip(mix2, p2))
        c2 = _chunks8(mix2)
        c3 = _chunks8(mix3)
        dst = pl.ds(_i32((g % 8) * 16), 16)
        for h in range(8):
            hidx[_i32(h), _i32(g // 8), dst] = _mod_p(c2 if h < 4 else c3, h)

    def fire(i):
        c, h = divmod(i, 8)
        r = i % _RING
        return pltpu.async_copy(
            ctab.at[hidx.at[_i32(h), _i32(c)]],
            ring.at[_i32(r)], ring_sems[r])

    def band_copy(i):
        c, h = divmod(i, 8)
        r = i % _RING
        band = pl.ds(_i32(h * _HEAD_DIM), _HEAD_DIM)

        def body(t, carry):
            obuf[_i32(c), t, band] = ring[_i32(r), t, band]
            return carry

        lax.fori_loop(0, 128, body, jnp.int32(0), unroll=4)

    def drain(entry):
        i, cp = entry
        cp.wait()
        band_copy(i)
        if i % 8 == 7:
            c = i // 8
            return pltpu.async_copy(
                obuf.at[_i32(c)],
                out.at[pl.ds(base + c * 128, 128)], sem)
        return None

    inflight = []
    writes = []
    for i in range(16):
        if len(inflight) == _RING:
            w = drain(inflight.pop(0))
            if w is not None:
                writes.append(w)
        inflight.append((i, fire(i)))
    while inflight:
        w = drain(inflight.pop(0))
        if w is not None:
            writes.append(w)
    for w in writes:
        w.wait()


def _ctab_body(*refs):
    x = jnp.concatenate([refs[h][...] for h in range(8)], axis=0)
    refs[8][...] = jnp.transpose(x, (1, 0))


def _build_ctab(*tts):
    grid = _CV // 2048
    in_specs = [pl.BlockSpec((16, 2048), lambda j: (jnp.int32(0), j)) for _ in range(8)]
    out_specs = pl.BlockSpec((2048, 128), lambda j: (j, jnp.int32(0)))
    return pl.pallas_call(
        _ctab_body,
        grid=(grid,),
        in_specs=in_specs,
        out_specs=out_specs,
        out_shape=jax.ShapeDtypeStruct((_CV, 128), jnp.float32),
    )(*tts)


@jax.jit
def _engram_call(inp, lut, mlimbs, ctab):
    mesh = plsc.VectorSubcoreMesh(core_axis_name="c", subcore_axis_name="s")
    f = functools.partial(
        pl.kernel,
        mesh=mesh,
        out_type=jax.ShapeDtypeStruct((_TOK, 8 * _HEAD_DIM), jnp.float32),
        scratch_types=[
            pltpu.VMEM((2, 128), jnp.int32),
            pltpu.VMEM((16,), jnp.int32),
            pltpu.VMEM((_WIN,), jnp.int32),
            pltpu.VMEM((8, 2, 128), jnp.int32),
            pltpu.VMEM((_RING, 128, 128), jnp.float32),
            pltpu.VMEM((2, 128, 128), jnp.float32),
            pltpu.VMEM((12, 16), jnp.int32),
            pltpu.SemaphoreType.DMA,
            pltpu.SemaphoreType.DMA,
            pltpu.SemaphoreType.DMA,
            pltpu.SemaphoreType.DMA,
            pltpu.SemaphoreType.DMA,
        ],
    )(_engram_body)
    return f(inp, lut, mlimbs, ctab)


def kernel(input_ids, lookup_table, multipliers,
           table_0, table_1, table_2, table_3,
           table_4, table_5, table_6, table_7):
    tables = (table_0, table_1, table_2, table_3,
              table_4, table_5, table_6, table_7)
    inp = input_ids.reshape(-1).astype(jnp.int32)
    lut = lookup_table.astype(jnp.int32)
    shifts = jnp.asarray([0, 16, 32, 48], dtype=multipliers.dtype)
    limbs = ((multipliers[:, None] >> shifts[None, :]) & 0xFFFF).astype(jnp.int32)
    mlimbs = jnp.broadcast_to(limbs.reshape(12, 1), (12, 16))
    ctab = _build_ctab(*(t.T for t in tables))
    out = _engram_call(inp, lut, mlimbs, ctab)
    return out.reshape(_B, _T, 8 * _HEAD_DIM)


# speedup vs baseline: 7.0253x; 1.1795x over previous
import functools

import jax
import jax.numpy as jnp
from jax import lax
from jax.experimental import pallas as pl
from jax.experimental.pallas import tpu as pltpu
from jax.experimental.pallas import tpu_sc as plsc

_PRIMES = (100003, 100019, 100043, 100049,
           100057, 100069, 100103, 100109)
_HEAD_DIM = 16
_TOKENIZER_VOCAB = 128000
_B = 4
_T = 2048
_TOK = _B * _T
_NW = 32
_CHUNK = _TOK // _NW
_LANES = 16
_GROUPS = _CHUNK // _LANES
_WIN = _CHUNK + 16
_CV = 102400
_RING = 4

_R8 = tuple(tuple(pow(2, 8 * k, p) for k in range(8)) for p in _PRIMES)


def _i32(v):
    return jnp.int32(v)


def _srl(x, k):
    return lax.shift_right_logical(x, jnp.int32(k))


def _prod_limbs(a, m):
    a0 = a & 0xFFFF
    a1 = _srl(a, 16)
    t = a0 * m[0]
    l0 = t & 0xFFFF
    c = _srl(t, 16)
    t = a0 * m[1] + a1 * m[0] + c
    l1 = t & 0xFFFF
    c = _srl(t, 16)
    t = a0 * m[2] + a1 * m[1] + c
    l2 = t & 0xFFFF
    c = _srl(t, 16)
    t = a0 * m[3] + a1 * m[2] + c
    l3 = t & 0xFFFF
    return (l0, l1, l2, l3)


def _chunks8(limbs):
    out = []
    for l in limbs:
        out.append(l & 0xFF)
        out.append(_srl(l, 8))
    return out


def _mod_p(chunks, h):
    p = _PRIMES[h]
    r8 = _R8[h]
    s = chunks[0] * r8[0]
    for k in range(1, 8):
        s = s + chunks[k] * r8[k]
    q = (s.astype(jnp.float32) * jnp.float32(1.0 / p)).astype(jnp.int32)
    r = s - q * p
    r = jnp.where(r < 0, r + p, r)
    r = jnp.where(r >= p, r - p, r)
    return r


def _engram_body(inp, lut, mlv_hbm, ctab,
                 out, raw_a, raw_b, comp, hidx, ring, obuf, mlv,
                 sem, rs0, rs1, rs2, rs3):
    ring_sems = (rs0, rs1, rs2, rs3)
    wid = lax.axis_index("s") * 2 + lax.axis_index("c")
    base = wid * _CHUNK
    start = base - 16

    pltpu.sync_copy(mlv_hbm, mlv)

    @pl.when(wid == 0)
    def _():
        raw_a[_i32(0), pl.ds(_i32(0), 16)] = jnp.zeros((16,), jnp.int32)
        pltpu.sync_copy(inp.at[pl.ds(_i32(0), 112)], raw_a.at[_i32(0), pl.ds(_i32(16), 112)])
        pltpu.sync_copy(inp.at[pl.ds(_i32(112), 128)], raw_a.at[_i32(1)])
        pltpu.sync_copy(inp.at[pl.ds(_i32(240), 16)], raw_b)

    @pl.when(wid > 0)
    def _():
        pltpu.sync_copy(inp.at[pl.ds(start, 128)], raw_a.at[_i32(0)])
        pltpu.sync_copy(inp.at[pl.ds(start + 128, 128)], raw_a.at[_i32(1)])
        pltpu.sync_copy(inp.at[pl.ds(start + 256, 16)], raw_b)

    for r in range(2):
        for j in range(8):
            sl = pl.ds(_i32(j * 16), 16)
            raw_a[_i32(r), sl] = jnp.clip(raw_a[_i32(r), sl], 0, _TOKENIZER_VOCAB - 1)
    raw_b[...] = jnp.clip(raw_b[...], 0, _TOKENIZER_VOCAB - 1)

    g1 = pltpu.async_copy(lut.at[raw_a.at[_i32(0)]], comp.at[pl.ds(_i32(0), 128)], sem)
    g2 = pltpu.async_copy(lut.at[raw_a.at[_i32(1)]], comp.at[pl.ds(_i32(128), 128)], sem)
    g3 = pltpu.async_copy(lut.at[raw_b], comp.at[pl.ds(_i32(256), 16)], sem)
    g1.wait()
    g2.wait()
    g3.wait()

    m0 = tuple(mlv[_i32(k)] for k in range(4))
    m1 = tuple(mlv[_i32(4 + k)] for k in range(4))
    m2 = tuple(mlv[_i32(8 + k)] for k in range(4))
    rowpos = (base & (_T - 1)) + lax.iota(jnp.int32, 16)

    for g in range(_GROUPS):
        off = 16 + g * 16
        s0v = comp[pl.ds(_i32(off), 16)]
        s1v = comp[pl.ds(_i32(off - 1), 16)]
        s2v = comp[pl.ds(_i32(off - 2), 16)]
        if g == 0:
            s1v = jnp.where(rowpos >= 1, s1v, 0)
            s2v = jnp.where(rowpos >= 2, s2v, 0)
        p0 = _prod_limbs(s0v, m0)
        p1 = _prod_limbs(s1v, m1)
        p2 = _prod_limbs(s2v, m2)
        mix2 = tuple(x ^ y for x, y in zip(p0, p1))
        mix3 = tuple(x ^ y for x, y in zip(mix2, p2))
        c2 = _chunks8(mix2)
        c3 = _chunks8(mix3)
        dst = pl.ds(_i32((g % 8) * 16), 16)
        for h in range(8):
            hidx[_i32(h), _i32(g // 8), dst] = _mod_p(c2 if h < 4 else c3, h)

    def fire(i):
        c, h = divmod(i, 8)
        r = i % _RING
        return pltpu.async_copy(
            ctab.at[hidx.at[_i32(h), _i32(c)]],
            ring.at[_i32(r)], ring_sems[r])

    def band_copy(i):
        c, h = divmod(i, 8)
        r = i % _RING
        band = pl.ds(_i32(h * _HEAD_DIM), _HEAD_DIM)

        def body(t, carry):
            obuf[_i32(c), t, band] = ring[_i32(r), t, band]
            return carry

        lax.fori_loop(0, 128, body, jnp.int32(0), unroll=4)

    def drain(entry):
        i, cp = entry
        cp.wait()
        band_copy(i)
        if i % 8 == 7:
            c = i // 8
            return pltpu.async_copy(
                obuf.at[_i32(c)],
                out.at[pl.ds(base + c * 128, 128)], sem)
        return None

    inflight = []
    writes = []
    for i in range(16):
        if len(inflight) == _RING:
            w = drain(inflight.pop(0))
            if w is not None:
                writes.append(w)
        inflight.append((i, fire(i)))
    while inflight:
        w = drain(inflight.pop(0))
        if w is not None:
            writes.append(w)
    for w in writes:
        w.wait()


def _ctab_body(*refs):
    x = jnp.concatenate([refs[h][...] for h in range(8)], axis=0)
    refs[8][...] = jnp.transpose(x, (1, 0))


def _build_ctab(*tts):
    grid = _CV // 4096
    in_specs = [pl.BlockSpec((16, 4096), lambda j: (jnp.int32(0), j)) for _ in range(8)]
    out_specs = pl.BlockSpec((4096, 128), lambda j: (j, jnp.int32(0)))
    return pl.pallas_call(
        _ctab_body,
        grid=(grid,),
        in_specs=in_specs,
        out_specs=out_specs,
        out_shape=jax.ShapeDtypeStruct((_CV, 128), jnp.float32),
    )(*tts)


@jax.jit
def _engram_call(inp, lut, mlimbs, ctab):
    mesh = plsc.VectorSubcoreMesh(core_axis_name="c", subcore_axis_name="s")
    f = functools.partial(
        pl.kernel,
        mesh=mesh,
        out_type=jax.ShapeDtypeStruct((_TOK, 8 * _HEAD_DIM), jnp.float32),
        scratch_types=[
            pltpu.VMEM((2, 128), jnp.int32),
            pltpu.VMEM((16,), jnp.int32),
            pltpu.VMEM((_WIN,), jnp.int32),
            pltpu.VMEM((8, 2, 128), jnp.int32),
            pltpu.VMEM((_RING, 128, 128), jnp.float32),
            pltpu.VMEM((2, 128, 128), jnp.float32),
            pltpu.VMEM((12, 16), jnp.int32),
            pltpu.SemaphoreType.DMA,
            pltpu.SemaphoreType.DMA,
            pltpu.SemaphoreType.DMA,
            pltpu.SemaphoreType.DMA,
            pltpu.SemaphoreType.DMA,
        ],
    )(_engram_body)
    return f(inp, lut, mlimbs, ctab)


def kernel(input_ids, lookup_table, multipliers,
           table_0, table_1, table_2, table_3,
           table_4, table_5, table_6, table_7):
    tables = (table_0, table_1, table_2, table_3,
              table_4, table_5, table_6, table_7)
    inp = input_ids.reshape(-1).astype(jnp.int32)
    lut = lookup_table.astype(jnp.int32)
    shifts = jnp.asarray([0, 16, 32, 48], dtype=multipliers.dtype)
    limbs = ((multipliers[:, None] >> shifts[None, :]) & 0xFFFF).astype(jnp.int32)
    mlimbs = jnp.broadcast_to(limbs.reshape(12, 1), (12, 16))
    ctab = _build_ctab(*(t.T for t in tables))
    out = _engram_call(inp, lut, mlimbs, ctab)
    return out.reshape(_B, _T, 8 * _HEAD_DIM)


# speedup vs baseline: 7.4431x; 1.0595x over previous
import functools

import jax
import jax.numpy as jnp
from jax import lax
from jax.experimental import pallas as pl
from jax.experimental.pallas import tpu as pltpu
from jax.experimental.pallas import tpu_sc as plsc

_PRIMES = (100003, 100019, 100043, 100049,
           100057, 100069, 100103, 100109)
_HEAD_DIM = 16
_TOKENIZER_VOCAB = 128000
_B = 4
_T = 2048
_TOK = _B * _T
_NW = 32
_CHUNK = _TOK // _NW
_LANES = 16
_GROUPS = _CHUNK // _LANES
_WIN = _CHUNK + 16
_CV = 106496
_RING = 4

_R8 = tuple(tuple(pow(2, 8 * k, p) for k in range(8)) for p in _PRIMES)


def _i32(v):
    return jnp.int32(v)


def _srl(x, k):
    return lax.shift_right_logical(x, jnp.int32(k))


def _prod_limbs(a, m):
    a0 = a & 0xFFFF
    a1 = _srl(a, 16)
    t = a0 * m[0]
    l0 = t & 0xFFFF
    c = _srl(t, 16)
    t = a0 * m[1] + a1 * m[0] + c
    l1 = t & 0xFFFF
    c = _srl(t, 16)
    t = a0 * m[2] + a1 * m[1] + c
    l2 = t & 0xFFFF
    c = _srl(t, 16)
    t = a0 * m[3] + a1 * m[2] + c
    l3 = t & 0xFFFF
    return (l0, l1, l2, l3)


def _chunks8(limbs):
    out = []
    for l in limbs:
        out.append(l & 0xFF)
        out.append(_srl(l, 8))
    return out


def _mod_p(chunks, h):
    p = _PRIMES[h]
    r8 = _R8[h]
    s = chunks[0] * r8[0]
    for k in range(1, 8):
        s = s + chunks[k] * r8[k]
    q = (s.astype(jnp.float32) * jnp.float32(1.0 / p)).astype(jnp.int32)
    r = s - q * p
    r = jnp.where(r < 0, r + p, r)
    r = jnp.where(r >= p, r - p, r)
    return r


def _engram_body(inp, lut, mlv_hbm, ctab,
                 out, raw_a, raw_b, comp, hidx, ring, obuf, mlv,
                 sem, rs0, rs1, rs2, rs3):
    ring_sems = (rs0, rs1, rs2, rs3)
    wid = lax.axis_index("s") * 2 + lax.axis_index("c")
    base = wid * _CHUNK
    start = base - 16

    pltpu.sync_copy(mlv_hbm, mlv)

    @pl.when(wid == 0)
    def _():
        raw_a[_i32(0), pl.ds(_i32(0), 16)] = jnp.zeros((16,), jnp.int32)
        pltpu.sync_copy(inp.at[pl.ds(_i32(0), 112)], raw_a.at[_i32(0), pl.ds(_i32(16), 112)])
        pltpu.sync_copy(inp.at[pl.ds(_i32(112), 128)], raw_a.at[_i32(1)])
        pltpu.sync_copy(inp.at[pl.ds(_i32(240), 16)], raw_b)

    @pl.when(wid > 0)
    def _():
        pltpu.sync_copy(inp.at[pl.ds(start, 128)], raw_a.at[_i32(0)])
        pltpu.sync_copy(inp.at[pl.ds(start + 128, 128)], raw_a.at[_i32(1)])
        pltpu.sync_copy(inp.at[pl.ds(start + 256, 16)], raw_b)

    for r in range(2):
        for j in range(8):
            sl = pl.ds(_i32(j * 16), 16)
            raw_a[_i32(r), sl] = jnp.clip(raw_a[_i32(r), sl], 0, _TOKENIZER_VOCAB - 1)
    raw_b[...] = jnp.clip(raw_b[...], 0, _TOKENIZER_VOCAB - 1)

    g1 = pltpu.async_copy(lut.at[raw_a.at[_i32(0)]], comp.at[pl.ds(_i32(0), 128)], sem)
    g2 = pltpu.async_copy(lut.at[raw_a.at[_i32(1)]], comp.at[pl.ds(_i32(128), 128)], sem)
    g3 = pltpu.async_copy(lut.at[raw_b], comp.at[pl.ds(_i32(256), 16)], sem)
    g1.wait()
    g2.wait()
    g3.wait()

    m0 = tuple(mlv[_i32(k)] for k in range(4))
    m1 = tuple(mlv[_i32(4 + k)] for k in range(4))
    m2 = tuple(mlv[_i32(8 + k)] for k in range(4))
    rowpos = (base & (_T - 1)) + lax.iota(jnp.int32, 16)

    for g in range(_GROUPS):
        off = 16 + g * 16
        s0v = comp[pl.ds(_i32(off), 16)]
        s1v = comp[pl.ds(_i32(off - 1), 16)]
        s2v = comp[pl.ds(_i32(off - 2), 16)]
        if g == 0:
            s1v = jnp.where(rowpos >= 1, s1v, 0)
            s2v = jnp.where(rowpos >= 2, s2v, 0)
        p0 = _prod_limbs(s0v, m0)
        p1 = _prod_limbs(s1v, m1)
        p2 = _prod_limbs(s2v, m2)
        mix2 = tuple(x ^ y for x, y in zip(p0, p1))
        mix3 = tuple(x ^ y for x, y in zip(mix2, p2))
        c2 = _chunks8(mix2)
        c3 = _chunks8(mix3)
        dst = pl.ds(_i32((g % 8) * 16), 16)
        for h in range(8):
            hidx[_i32(h), _i32(g // 8), dst] = _mod_p(c2 if h < 4 else c3, h)

    def fire(i):
        c, h = divmod(i, 8)
        r = i % _RING
        return pltpu.async_copy(
            ctab.at[hidx.at[_i32(h), _i32(c)]],
            ring.at[_i32(r)], ring_sems[r])

    def band_copy(i):
        c, h = divmod(i, 8)
        r = i % _RING
        band = pl.ds(_i32(h * _HEAD_DIM), _HEAD_DIM)

        def body(t, carry):
            obuf[_i32(c), t, band] = ring[_i32(r), t, band]
            return carry

        lax.fori_loop(0, 128, body, jnp.int32(0), unroll=4)

    def drain(entry):
        i, cp = entry
        cp.wait()
        band_copy(i)
        if i % 8 == 7:
            c = i // 8
            return pltpu.async_copy(
                obuf.at[_i32(c)],
                out.at[pl.ds(base + c * 128, 128)], sem)
        return None

    inflight = []
    writes = []
    for i in range(16):
        if len(inflight) == _RING:
            w = drain(inflight.pop(0))
            if w is not None:
                writes.append(w)
        inflight.append((i, fire(i)))
    while inflight:
        w = drain(inflight.pop(0))
        if w is not None:
            writes.append(w)
    for w in writes:
        w.wait()


def _ctab_body(*refs):
    x = jnp.concatenate([refs[h][...] for h in range(8)], axis=0)
    refs[8][...] = jnp.transpose(x, (1, 0))


def _build_ctab(*tts):
    grid = _CV // 8192
    in_specs = [pl.BlockSpec((16, 8192), lambda j: (jnp.int32(0), j)) for _ in range(8)]
    out_specs = pl.BlockSpec((8192, 128), lambda j: (j, jnp.int32(0)))
    return pl.pallas_call(
        _ctab_body,
        grid=(grid,),
        in_specs=in_specs,
        out_specs=out_specs,
        out_shape=jax.ShapeDtypeStruct((_CV, 128), jnp.float32),
    )(*tts)


@jax.jit
def _engram_call(inp, lut, mlimbs, ctab):
    mesh = plsc.VectorSubcoreMesh(core_axis_name="c", subcore_axis_name="s")
    f = functools.partial(
        pl.kernel,
        mesh=mesh,
        out_type=jax.ShapeDtypeStruct((_TOK, 8 * _HEAD_DIM), jnp.float32),
        scratch_types=[
            pltpu.VMEM((2, 128), jnp.int32),
            pltpu.VMEM((16,), jnp.int32),
            pltpu.VMEM((_WIN,), jnp.int32),
            pltpu.VMEM((8, 2, 128), jnp.int32),
            pltpu.VMEM((_RING, 128, 128), jnp.float32),
            pltpu.VMEM((2, 128, 128), jnp.float32),
            pltpu.VMEM((12, 16), jnp.int32),
            pltpu.SemaphoreType.DMA,
            pltpu.SemaphoreType.DMA,
            pltpu.SemaphoreType.DMA,
            pltpu.SemaphoreType.DMA,
            pltpu.SemaphoreType.DMA,
        ],
    )(_engram_body)
    return f(inp, lut, mlimbs, ctab)


def kernel(input_ids, lookup_table, multipliers,
           table_0, table_1, table_2, table_3,
           table_4, table_5, table_6, table_7):
    tables = (table_0, table_1, table_2, table_3,
              table_4, table_5, table_6, table_7)
    inp = input_ids.reshape(-1).astype(jnp.int32)
    lut = lookup_table.astype(jnp.int32)
    shifts = jnp.asarray([0, 16, 32, 48], dtype=multipliers.dtype)
    limbs = ((multipliers[:, None] >> shifts[None, :]) & 0xFFFF).astype(jnp.int32)
    mlimbs = jnp.broadcast_to(limbs.reshape(12, 1), (12, 16))
    ctab = _build_ctab(*(t.T for t in tables))
    out = _engram_call(inp, lut, mlimbs, ctab)
    return out.reshape(_B, _T, 8 * _HEAD_DIM)


# speedup vs baseline: 7.6717x; 1.0307x over previous
import functools

import jax
import jax.numpy as jnp
from jax import lax
from jax.experimental import pallas as pl
from jax.experimental.pallas import tpu as pltpu
from jax.experimental.pallas import tpu_sc as plsc

_PRIMES = (100003, 100019, 100043, 100049,
           100057, 100069, 100103, 100109)
_HEAD_DIM = 16
_TOKENIZER_VOCAB = 128000
_B = 4
_T = 2048
_TOK = _B * _T
_NW = 32
_CHUNK = _TOK // _NW
_LANES = 16
_GROUPS = _CHUNK // _LANES
_WIN = _CHUNK + 16
_CV = 106496
_RING = 4

_R8 = tuple(tuple(pow(2, 8 * k, p) for k in range(8)) for p in _PRIMES)


def _i32(v):
    return jnp.int32(v)


def _srl(x, k):
    return lax.shift_right_logical(x, jnp.int32(k))


def _prod_limbs(a, m):
    a0 = a & 0xFFFF
    a1 = _srl(a, 16)
    t = a0 * m[0]
    l0 = t & 0xFFFF
    c = _srl(t, 16)
    t = a0 * m[1] + a1 * m[0] + c
    l1 = t & 0xFFFF
    c = _srl(t, 16)
    t = a0 * m[2] + a1 * m[1] + c
    l2 = t & 0xFFFF
    c = _srl(t, 16)
    t = a0 * m[3] + a1 * m[2] + c
    l3 = t & 0xFFFF
    return (l0, l1, l2, l3)


def _chunks8(limbs):
    out = []
    for l in limbs:
        out.append(l & 0xFF)
        out.append(_srl(l, 8))
    return out


def _mod_p(chunks, h):
    p = _PRIMES[h]
    r8 = _R8[h]
    s = chunks[0] * r8[0]
    for k in range(1, 8):
        s = s + chunks[k] * r8[k]
    q = (s.astype(jnp.float32) * jnp.float32(1.0 / p)).astype(jnp.int32)
    r = s - q * p
    r = jnp.where(r < 0, r + p, r)
    r = jnp.where(r >= p, r - p, r)
    return r


def _hash_body(inp, lut, mlv_hbm,
               hidx_out, raw_a, raw_b, comp, hidx, mlv, sem):
    wid = lax.axis_index("s") * 2 + lax.axis_index("c")
    base = wid * _CHUNK
    start = base - 16

    pltpu.sync_copy(mlv_hbm, mlv)

    @pl.when(wid == 0)
    def _():
        raw_a[_i32(0), pl.ds(_i32(0), 16)] = jnp.zeros((16,), jnp.int32)
        pltpu.sync_copy(inp.at[pl.ds(_i32(0), 112)], raw_a.at[_i32(0), pl.ds(_i32(16), 112)])
        pltpu.sync_copy(inp.at[pl.ds(_i32(112), 128)], raw_a.at[_i32(1)])
        pltpu.sync_copy(inp.at[pl.ds(_i32(240), 16)], raw_b)

    @pl.when(wid > 0)
    def _():
        pltpu.sync_copy(inp.at[pl.ds(start, 128)], raw_a.at[_i32(0)])
        pltpu.sync_copy(inp.at[pl.ds(start + 128, 128)], raw_a.at[_i32(1)])
        pltpu.sync_copy(inp.at[pl.ds(start + 256, 16)], raw_b)

    for r in range(2):
        for j in range(8):
            sl = pl.ds(_i32(j * 16), 16)
            raw_a[_i32(r), sl] = jnp.clip(raw_a[_i32(r), sl], 0, _TOKENIZER_VOCAB - 1)
    raw_b[...] = jnp.clip(raw_b[...], 0, _TOKENIZER_VOCAB - 1)

    g1 = pltpu.async_copy(lut.at[raw_a.at[_i32(0)]], comp.at[pl.ds(_i32(0), 128)], sem)
    g2 = pltpu.async_copy(lut.at[raw_a.at[_i32(1)]], comp.at[pl.ds(_i32(128), 128)], sem)
    g3 = pltpu.async_copy(lut.at[raw_b], comp.at[pl.ds(_i32(256), 16)], sem)
    g1.wait()
    g2.wait()
    g3.wait()

    m0 = tuple(mlv[_i32(k)] for k in range(4))
    m1 = tuple(mlv[_i32(4 + k)] for k in range(4))
    m2 = tuple(mlv[_i32(8 + k)] for k in range(4))
    rowpos = (base & (_T - 1)) + lax.iota(jnp.int32, 16)

    for g in range(_GROUPS):
        off = 16 + g * 16
        s0v = comp[pl.ds(_i32(off), 16)]
        s1v = comp[pl.ds(_i32(off - 1), 16)]
        s2v = comp[pl.ds(_i32(off - 2), 16)]
        if g == 0:
            s1v = jnp.where(rowpos >= 1, s1v, 0)
            s2v = jnp.where(rowpos >= 2, s2v, 0)
        p0 = _prod_limbs(s0v, m0)
        p1 = _prod_limbs(s1v, m1)
        p2 = _prod_limbs(s2v, m2)
        mix2 = tuple(x ^ y for x, y in zip(p0, p1))
        mix3 = tuple(x ^ y for x, y in zip(mix2, p2))
        c2 = _chunks8(mix2)
        c3 = _chunks8(mix3)
        dst = pl.ds(_i32((g % 8) * 16), 16)
        for h in range(8):
            hidx[_i32(h), _i32(g // 8), dst] = _mod_p(c2 if h < 4 else c3, h)

    pltpu.sync_copy(hidx, hidx_out.at[wid])


def _gather_body(hidx_hbm, ctab,
                 out, hidx, ring, obuf,
                 sem, rs0, rs1, rs2, rs3):
    ring_sems = (rs0, rs1, rs2, rs3)
    wid = lax.axis_index("s") * 2 + lax.axis_index("c")
    base = wid * _CHUNK
    pltpu.sync_copy(hidx_hbm.at[wid], hidx)

    def fire(i):
        c, h = divmod(i, 8)
        r = i % _RING
        return pltpu.async_copy(
            ctab.at[hidx.at[_i32(h), _i32(c)]],
            ring.at[_i32(r)], ring_sems[r])

    def band_copy(i):
        c, h = divmod(i, 8)
        r = i % _RING
        band = pl.ds(_i32(h * _HEAD_DIM), _HEAD_DIM)

        def body(t, carry):
            obuf[_i32(c), t, band] = ring[_i32(r), t, band]
            return carry

        lax.fori_loop(0, 128, body, jnp.int32(0), unroll=4)

    def drain(entry):
        i, cp = entry
        cp.wait()
        band_copy(i)
        if i % 8 == 7:
            c = i // 8
            return pltpu.async_copy(
                obuf.at[_i32(c)],
                out.at[pl.ds(base + c * 128, 128)], sem)
        return None

    inflight = []
    writes = []
    for i in range(16):
        if len(inflight) == _RING:
            w = drain(inflight.pop(0))
            if w is not None:
                writes.append(w)
        inflight.append((i, fire(i)))
    while inflight:
        w = drain(inflight.pop(0))
        if w is not None:
            writes.append(w)
    for w in writes:
        w.wait()


def _ctab_body(*refs):
    x = jnp.concatenate([refs[h][...] for h in range(8)], axis=0)
    refs[8][...] = jnp.transpose(x, (1, 0))


def _build_ctab(*tts):
    grid = _CV // 8192
    in_specs = [pl.BlockSpec((16, 8192), lambda j: (jnp.int32(0), j)) for _ in range(8)]
    out_specs = pl.BlockSpec((8192, 128), lambda j: (j, jnp.int32(0)))
    return pl.pallas_call(
        _ctab_body,
        grid=(grid,),
        in_specs=in_specs,
        out_specs=out_specs,
        out_shape=jax.ShapeDtypeStruct((_CV, 128), jnp.float32),
    )(*tts)


@jax.jit
def _engram_call(inp, lut, mlimbs, ctab):
    mesh = plsc.VectorSubcoreMesh(core_axis_name="c", subcore_axis_name="s")
    hash_f = functools.partial(
        pl.kernel,
        mesh=mesh,
        out_type=jax.ShapeDtypeStruct((_NW, 8, 2, 128), jnp.int32),
        scratch_types=[
            pltpu.VMEM((2, 128), jnp.int32),
            pltpu.VMEM((16,), jnp.int32),
            pltpu.VMEM((_WIN,), jnp.int32),
            pltpu.VMEM((8, 2, 128), jnp.int32),
            pltpu.VMEM((12, 16), jnp.int32),
            pltpu.SemaphoreType.DMA,
        ],
    )(_hash_body)
    hidx = hash_f(inp, lut, mlimbs)
    gather_f = functools.partial(
        pl.kernel,
        mesh=mesh,
        out_type=jax.ShapeDtypeStruct((_TOK, 8 * _HEAD_DIM), jnp.float32),
        scratch_types=[
            pltpu.VMEM((8, 2, 128), jnp.int32),
            pltpu.VMEM((_RING, 128, 128), jnp.float32),
            pltpu.VMEM((2, 128, 128), jnp.float32),
            pltpu.SemaphoreType.DMA,
            pltpu.SemaphoreType.DMA,
            pltpu.SemaphoreType.DMA,
            pltpu.SemaphoreType.DMA,
            pltpu.SemaphoreType.DMA,
        ],
    )(_gather_body)
    return gather_f(hidx, ctab)


def kernel(input_ids, lookup_table, multipliers,
           table_0, table_1, table_2, table_3,
           table_4, table_5, table_6, table_7):
    tables = (table_0, table_1, table_2, table_3,
              table_4, table_5, table_6, table_7)
    inp = input_ids.reshape(-1).astype(jnp.int32)
    lut = lookup_table.astype(jnp.int32)
    shifts = jnp.asarray([0, 16, 32, 48], dtype=multipliers.dtype)
    limbs = ((multipliers[:, None] >> shifts[None, :]) & 0xFFFF).astype(jnp.int32)
    mlimbs = jnp.broadcast_to(limbs.reshape(12, 1), (12, 16))
    ctab = _build_ctab(*(t.T for t in tables))
    out = _engram_call(inp, lut, mlimbs, ctab)
    return out.reshape(_B, _T, 8 * _HEAD_DIM)
